# baseline (speedup 1.0000x reference)
import functools

import jax
import jax.numpy as jnp
from jax import lax
from jax.experimental import pallas as pl
from jax.experimental.pallas import tpu as pltpu
from jax.experimental.pallas import tpu_sc as plsc

VOCAB = 1000000
B = 4096
D = 64
LSEG = 50
NC = 2
NS = 16
NW = NC * NS
BAGS_W = B // NW
CHUNK = 4
NCHUNK = BAGS_W // CHUNK
TOK_C = CHUNK * LSEG
TOK_PAD = 224
GS = 40
NSUB = TOK_C // GS
INV_L = 1.0 / LSEG
TBLK = 8192
NBLK = 62
SPLIT = NBLK * TBLK


def _stage_body(lo_ref, hi_ref, o_ref):
    o_ref[:, 0:D] = lo_ref[...].T
    o_ref[:, D:128] = hi_ref[...].T


def _stage(tableT):
    return pl.pallas_call(
        _stage_body,
        grid=(NBLK,),
        in_specs=[
            pl.BlockSpec((D, TBLK), lambda i: (0, i)),
            pl.BlockSpec(
                (D, TBLK),
                lambda i: (0, jnp.minimum(i + NBLK, VOCAB // TBLK))),
        ],
        out_specs=pl.BlockSpec((TBLK, 128), lambda i: (i, 0)),
        out_shape=jax.ShapeDtypeStruct((SPLIT, 128), jnp.float32),
    )(tableT, tableT)


def _sc_pool_body(text_h, staged_h, pooled_h, tok_v0, tok_v1, idx_v0,
                  idx_v1, rows_v0, rows_v1, pool_v, sem0, sem1):
    c = lax.axis_index("c")
    s = lax.axis_index("s")
    wid = s * NC + c
    toks = (tok_v0, tok_v1)
    idxs = (idx_v0, idx_v1)
    rows = (rows_v0, rows_v1)
    sems = (sem0, sem1)

    def fire(ch, bi):
        tok0 = (wid * BAGS_W + ch * CHUNK) * LSEG
        pltpu.sync_copy(text_h.at[pl.ds(tok0, TOK_C)],
                        toks[bi].at[pl.ds(0, TOK_C)])
        for i in range(TOK_PAD // 16):
            tv = toks[bi][pl.ds(i * 16, 16)]
            idxs[bi][pl.ds(i * 16, 16)] = tv - jnp.where(
                tv >= SPLIT, SPLIT, 0)
        for g in range(NSUB):
            pltpu.make_async_copy(
                staged_h.at[idxs[bi].at[pl.ds(g * GS, GS)]],
                rows[bi].at[pl.ds(g * GS, GS)],
                sems[bi],
            ).start()

    def drain(bi):
        for g in range(NSUB):
            pltpu.make_async_copy(
                staged_h.at[idxs[bi].at[pl.ds(g * GS, GS)]],
                rows[bi].at[pl.ds(g * GS, GS)],
                sems[bi],
            ).wait()

    def reduce(ch, bi):
        tv_ref = toks[bi]
        rv = rows[bi]
        bag0 = wid * BAGS_W + ch * CHUNK
        zero = jnp.zeros((16,), jnp.float32)
        for cc in range(CHUNK):
            r0 = cc * LSEG
            a0 = a1 = a2 = a3 = zero
            for g0 in (0, 16, 32, 48):
                glen = 16 if g0 < 48 else LSEG - 48
                par = jnp.where(tv_ref[pl.ds(r0 + g0, 16)] >= SPLIT, D, 0)
                for u in range(glen):
                    off = par[u]
                    r = r0 + g0 + u
                    a0 = a0 + rv[r, pl.ds(off, 16)]
                    a1 = a1 + rv[r, pl.ds(off + 16, 16)]
                    a2 = a2 + rv[r, pl.ds(off + 32, 16)]
                    a3 = a3 + rv[r, pl.ds(off + 48, 16)]
            pool_v[cc, pl.ds(0, 16)] = a0 * INV_L
            pool_v[cc, pl.ds(16, 16)] = a1 * INV_L
            pool_v[cc, pl.ds(32, 16)] = a2 * INV_L
            pool_v[cc, pl.ds(48, 16)] = a3 * INV_L
        pltpu.sync_copy(pool_v, pooled_h.at[pl.ds(bag0, CHUNK)])

    fire(0, 0)

    def pair_body(p, carry):
        ch = p * 2
        fire(ch + 1, 1)
        drain(0)
        reduce(ch, 0)
        fire(jnp.minimum(ch + 2, NCHUNK - 1), 0)
        drain(1)
        reduce(ch + 1, 1)
        return carry

    lax.fori_loop(0, NCHUNK // 2, pair_body, 0)
    drain(0)


_sc_pool = functools.partial(
    pl.kernel,
    out_type=jax.ShapeDtypeStruct((B, D), jnp.float32),
    mesh=plsc.VectorSubcoreMesh(core_axis_name="c", subcore_axis_name="s"),
    scratch_types=[
        pltpu.VMEM((TOK_PAD,), jnp.int32),
        pltpu.VMEM((TOK_PAD,), jnp.int32),
        pltpu.VMEM((TOK_PAD,), jnp.int32),
        pltpu.VMEM((TOK_PAD,), jnp.int32),
        pltpu.VMEM((TOK_C, 128), jnp.float32),
        pltpu.VMEM((TOK_C, 128), jnp.float32),
        pltpu.VMEM((CHUNK, D), jnp.float32),
        pltpu.SemaphoreType.DMA,
        pltpu.SemaphoreType.DMA,
    ],
)(_sc_pool_body)


def _mlp_body(x_ref, w1_ref, b1_ref, w2_ref, b2_ref, w3_ref, b3_ref,
              w4_ref, b4_ref, o_ref):
    dot = lambda a, b: lax.dot_general(
        a, b, (((1,), (1,)), ((), ())),
        preferred_element_type=jnp.float32,
        precision=lax.Precision.HIGHEST,
    )
    h = jnp.maximum(dot(x_ref[...], w1_ref[...]) + b1_ref[...], 0.0)
    h = jnp.maximum(dot(h, w2_ref[...]) + b2_ref[...], 0.0)
    h = dot(h, w3_ref[...]) + b3_ref[...]
    o_ref[...] = dot(h, w4_ref[...]) + b4_ref[...]


def _mlp(pooled, W1, b1, W2, b2, W3, b3, W4, b4):
    bm = 512
    grid = (B // bm,)
    full = lambda shape: pl.BlockSpec(shape, lambda i: (0,) * len(shape))
    return pl.pallas_call(
        _mlp_body,
        grid=grid,
        in_specs=[
            pl.BlockSpec((bm, D), lambda i: (i, 0)),
            full(W1.shape), full(b1.shape),
            full(W2.shape), full(b2.shape),
            full(W3.shape), full(b3.shape),
            full(W4.shape), full(b4.shape),
        ],
        out_specs=pl.BlockSpec((bm, W4.shape[0]), lambda i: (i, 0)),
        out_shape=jax.ShapeDtypeStruct((B, W4.shape[0]), jnp.float32),
    )(pooled, W1, b1, W2, b2, W3, b3, W4, b4)


def kernel(text, offsets, table, W1, b1, W2, b2, W3, b3, W4, b4):
    del offsets
    staged = _stage(table.T)
    pooled = _sc_pool(text, staged)
    return _mlp(pooled, W1, b1, W2, b2, W3, b3, W4, b4)


# speedup vs baseline: 1.0709x; 1.0709x over previous
import functools

import jax
import jax.numpy as jnp
from jax import lax
from jax.experimental import pallas as pl
from jax.experimental.pallas import tpu as pltpu
from jax.experimental.pallas import tpu_sc as plsc

VOCAB = 1000000
B = 4096
D = 64
LSEG = 50
NC = 2
NS = 16
NW = NC * NS
BAGS_W = B // NW
CHUNK = 8
NCHUNK = BAGS_W // CHUNK
TOK_C = CHUNK * LSEG
GS = 80
NSUB = TOK_C // GS
INV_L = 1.0 / LSEG
TBLK = 32768


def _stage_body(xt_ref, o_ref):
    o_ref[:, 0:D] = xt_ref[...].T


def _stage(tableT):
    grid = (pl.cdiv(VOCAB, TBLK),)
    return pl.pallas_call(
        _stage_body,
        grid=grid,
        in_specs=[pl.BlockSpec((D, TBLK), lambda i: (0, i))],
        out_specs=pl.BlockSpec((TBLK, 128), lambda i: (i, 0)),
        out_shape=jax.ShapeDtypeStruct((VOCAB, 128), jnp.float32),
        compiler_params=pltpu.CompilerParams(
            vmem_limit_bytes=100 * 1024 * 1024),
    )(tableT)


def _sc_pool_body(text_h, staged_h, pooled_h, tok_v0, tok_v1, rows_v0,
                  rows_v1, pool_v, sem0, sem1):
    c = lax.axis_index("c")
    s = lax.axis_index("s")
    wid = s * NC + c
    toks = (tok_v0, tok_v1)
    rows = (rows_v0, rows_v1)
    sems = (sem0, sem1)

    def fire(ch, bi):
        tok0 = (wid * BAGS_W + ch * CHUNK) * LSEG
        pltpu.sync_copy(text_h.at[pl.ds(tok0, TOK_C)], toks[bi])
        for g in range(NSUB):
            pltpu.make_async_copy(
                staged_h.at[toks[bi].at[pl.ds(g * GS, GS)]],
                rows[bi].at[pl.ds(g * GS, GS)],
                sems[bi],
            ).start()

    def drain(bi):
        for g in range(NSUB):
            pltpu.make_async_copy(
                staged_h.at[toks[bi].at[pl.ds(g * GS, GS)]],
                rows[bi].at[pl.ds(g * GS, GS)],
                sems[bi],
            ).wait()

    def reduce(ch, bi):
        rv = rows[bi]
        bag0 = wid * BAGS_W + ch * CHUNK
        zero = jnp.zeros((16,), jnp.float32)
        for cc in range(CHUNK):
            r0 = cc * LSEG

            def t_body(i, accs, rv=rv, r0=r0):
                a0, a1, a2, a3 = accs
                r = r0 + i * 5
                for u in range(5):
                    a0 = a0 + rv[r + u, pl.ds(0, 16)]
                    a1 = a1 + rv[r + u, pl.ds(16, 16)]
                    a2 = a2 + rv[r + u, pl.ds(32, 16)]
                    a3 = a3 + rv[r + u, pl.ds(48, 16)]
                return (a0, a1, a2, a3)

            a0, a1, a2, a3 = lax.fori_loop(0, LSEG // 5, t_body,
                                           (zero, zero, zero, zero))
            pool_v[cc, pl.ds(0, 16)] = a0 * INV_L
            pool_v[cc, pl.ds(16, 16)] = a1 * INV_L
            pool_v[cc, pl.ds(32, 16)] = a2 * INV_L
            pool_v[cc, pl.ds(48, 16)] = a3 * INV_L
        pltpu.sync_copy(pool_v, pooled_h.at[pl.ds(bag0, CHUNK)])

    fire(0, 0)

    def pair_body(p, carry):
        ch = p * 2
        fire(ch + 1, 1)
        drain(0)
        reduce(ch, 0)
        fire(ch + 2, 0)
        drain(1)
        reduce(ch + 1, 1)
        return carry

    lax.fori_loop(0, NCHUNK // 2 - 1, pair_body, 0)
    fire(NCHUNK - 1, 1)
    drain(0)
    reduce(NCHUNK - 2, 0)
    drain(1)
    reduce(NCHUNK - 1, 1)


_sc_pool = functools.partial(
    pl.kernel,
    out_type=jax.ShapeDtypeStruct((B, D), jnp.float32),
    mesh=plsc.VectorSubcoreMesh(core_axis_name="c", subcore_axis_name="s"),
    scratch_types=[
        pltpu.VMEM((TOK_C,), jnp.int32),
        pltpu.VMEM((TOK_C,), jnp.int32),
        pltpu.VMEM((TOK_C, 128), jnp.float32),
        pltpu.VMEM((TOK_C, 128), jnp.float32),
        pltpu.VMEM((CHUNK, D), jnp.float32),
        pltpu.SemaphoreType.DMA,
        pltpu.SemaphoreType.DMA,
    ],
)(_sc_pool_body)


def _mlp_body(x_ref, w1_ref, b1_ref, w2_ref, b2_ref, w3_ref, b3_ref,
              w4_ref, b4_ref, o_ref):
    dot = lambda a, b: lax.dot_general(
        a, b, (((1,), (1,)), ((), ())),
        preferred_element_type=jnp.float32,
        precision=lax.Precision.HIGHEST,
    )
    h = jnp.maximum(dot(x_ref[...], w1_ref[...]) + b1_ref[...], 0.0)
    h = jnp.maximum(dot(h, w2_ref[...]) + b2_ref[...], 0.0)
    h = dot(h, w3_ref[...]) + b3_ref[...]
    o_ref[...] = dot(h, w4_ref[...]) + b4_ref[...]


def _mlp(pooled, W1, b1, W2, b2, W3, b3, W4, b4):
    bm = 512
    grid = (B // bm,)
    full = lambda shape: pl.BlockSpec(shape, lambda i: (0,) * len(shape))
    return pl.pallas_call(
        _mlp_body,
        grid=grid,
        in_specs=[
            pl.BlockSpec((bm, D), lambda i: (i, 0)),
            full(W1.shape), full(b1.shape),
            full(W2.shape), full(b2.shape),
            full(W3.shape), full(b3.shape),
            full(W4.shape), full(b4.shape),
        ],
        out_specs=pl.BlockSpec((bm, W4.shape[0]), lambda i: (i, 0)),
        out_shape=jax.ShapeDtypeStruct((B, W4.shape[0]), jnp.float32),
    )(pooled, W1, b1, W2, b2, W3, b3, W4, b4)


def kernel(text, offsets, table, W1, b1, W2, b2, W3, b3, W4, b4):
    del offsets
    staged = _stage(table.T)
    pooled = _sc_pool(text, staged)
    return _mlp(pooled, W1, b1, W2, b2, W3, b3, W4, b4)
